# baseline (speedup 1.0000x reference)
import functools

import jax
import jax.numpy as jnp
from jax import lax
from jax.experimental import pallas as pl
from jax.experimental.pallas import tpu as pltpu
from jax.experimental.pallas import tpu_sc as plsc

_SC_CORES = 2
_SC_SUBCORES = 16
_NW = _SC_CORES * _SC_SUBCORES


def _routing(species, N, E, BM, G):
    R = G * BM
    perm = jnp.argsort(species).astype(jnp.int32)
    counts = jnp.bincount(species, length=E).astype(jnp.int32)
    nblk = (counts + BM - 1) // BM
    blk_end = jnp.cumsum(nblk)
    blk_start = blk_end - nblk
    tok_start = jnp.cumsum(counts) - counts
    g = jnp.arange(G, dtype=jnp.int32)
    e_of_g = jnp.searchsorted(blk_end, g, side="right").astype(jnp.int32)
    e_of_g = jnp.minimum(e_of_g, E - 1)
    i = jnp.arange(BM, dtype=jnp.int32)
    r = (g[:, None] - blk_start[e_of_g][:, None]) * BM + i[None, :]
    tok = tok_start[e_of_g][:, None] + r
    valid = r < counts[e_of_g][:, None]
    gathered_tok = perm[jnp.clip(tok, 0, N - 1)]
    src = jnp.where(valid, gathered_tok, 0).reshape(-1)
    slots = jnp.arange(R, dtype=jnp.int32)
    scat_to = jnp.where(valid.reshape(-1), gathered_tok.reshape(-1), N)
    pos = jnp.zeros((N,), jnp.int32).at[scat_to].set(slots, mode="drop")
    return e_of_g, src, pos


def _chunk_rows(b_per_w, row_bytes, budget=230 * 1024):
    cmax = max(8, min(b_per_w, budget // row_bytes))
    for c in range(cmax - cmax % 8, 0, -8):
        if b_per_w % c == 0:
            return c
    return 8


def _sc_gather(table, idx, R):
    _, D = table.shape
    b_per_w = R // _NW
    C = _chunk_rows(b_per_w, D * 4)
    NCH = b_per_w // C
    mesh = plsc.VectorSubcoreMesh(core_axis_name="c", subcore_axis_name="s")

    @functools.partial(
        pl.kernel,
        mesh=mesh,
        out_type=jax.ShapeDtypeStruct((R, D), table.dtype),
        scratch_types=[
            pltpu.VMEM((C,), jnp.int32),
            pltpu.VMEM((C,), jnp.int32),
            pltpu.VMEM((C, D), table.dtype),
            pltpu.VMEM((C, D), table.dtype),
            pltpu.SemaphoreType.DMA,
            pltpu.SemaphoreType.DMA,
        ],
    )
    def gather_k(table_hbm, idx_hbm, out_hbm, i0, i1, r0, r1, gsem, wsem):
        wid = lax.axis_index("s") * _SC_CORES + lax.axis_index("c")
        base = wid * b_per_w
        idx_v = (i0, i1)
        rows_v = (r0, r1)
        gathers = [None] * NCH
        writes = [None] * NCH
        pltpu.sync_copy(idx_hbm.at[pl.ds(base, C)], idx_v[0])
        gathers[0] = pltpu.async_copy(table_hbm.at[idx_v[0]], rows_v[0], gsem)
        for j in range(NCH):
            if j + 1 < NCH:
                if j >= 1:
                    writes[j - 1].wait()
                pltpu.sync_copy(
                    idx_hbm.at[pl.ds(base + (j + 1) * C, C)], idx_v[(j + 1) % 2]
                )
                gathers[j + 1] = pltpu.async_copy(
                    table_hbm.at[idx_v[(j + 1) % 2]], rows_v[(j + 1) % 2], gsem
                )
            gathers[j].wait()
            writes[j] = pltpu.async_copy(
                rows_v[j % 2], out_hbm.at[pl.ds(base + j * C, C)], wsem
            )
        for j in range(max(0, NCH - 2), NCH):
            writes[j].wait()

    return gather_k(table, idx)


def _mlp_body(e_ref, x_ref, w1_ref, b1_ref, w2_ref, b2_ref, y_ref):
    xb = x_ref[...].astype(jnp.bfloat16)
    h = jnp.dot(xb, w1_ref[0], preferred_element_type=jnp.float32)
    h = jax.nn.silu(h + b1_ref[0, 0])
    y = jnp.dot(h.astype(jnp.bfloat16), w2_ref[0], preferred_element_type=jnp.float32)
    y_ref[...] = y + b2_ref[0, 0]


def _grouped_mlp(e_of_g, x, W1, b1, W2, b2, BM, G):
    R = x.shape[0]
    D_IN, D_H = W1.shape[1], W1.shape[2]
    D_OUT = W2.shape[2]
    grid_spec = pltpu.PrefetchScalarGridSpec(
        num_scalar_prefetch=1,
        grid=(G,),
        in_specs=[
            pl.BlockSpec((BM, D_IN), lambda g, e: (g, 0)),
            pl.BlockSpec((1, D_IN, D_H), lambda g, e: (e[g], 0, 0)),
            pl.BlockSpec((1, 1, D_H), lambda g, e: (e[g], 0, 0)),
            pl.BlockSpec((1, D_H, D_OUT), lambda g, e: (e[g], 0, 0)),
            pl.BlockSpec((1, 1, D_OUT), lambda g, e: (e[g], 0, 0)),
        ],
        out_specs=pl.BlockSpec((BM, D_OUT), lambda g, e: (g, 0)),
    )
    return pl.pallas_call(
        _mlp_body,
        grid_spec=grid_spec,
        out_shape=jax.ShapeDtypeStruct((R, D_OUT), jnp.float32),
    )(e_of_g, x, W1, b1[:, None, :], W2, b2[:, None, :])


def kernel(density, species, W1, b1, W2, b2):
    N, D_IN = density.shape
    E = W1.shape[0]
    BM = 128
    G = N // BM + E
    R = G * BM

    e_of_g, src, pos = _routing(species, N, E, BM, G)
    gathered = _sc_gather(density, src, R)
    y = _grouped_mlp(
        e_of_g,
        gathered,
        W1.astype(jnp.bfloat16),
        b1,
        W2.astype(jnp.bfloat16),
        b2,
        BM,
        G,
    )
    return _sc_gather(y, pos, N)


# speedup vs baseline: 1.0805x; 1.0805x over previous
import functools

import jax
import jax.numpy as jnp
from jax import lax
from jax.experimental import pallas as pl
from jax.experimental.pallas import tpu as pltpu
from jax.experimental.pallas import tpu_sc as plsc

_SC_CORES = 2
_SC_SUBCORES = 16
_NW = _SC_CORES * _SC_SUBCORES


def _routing(species, N, E, BM, G):
    R = G * BM
    one_hot = (species[:, None] == jnp.arange(E, dtype=species.dtype)).astype(
        jnp.int32
    )
    counts = one_hot.sum(axis=0)
    within = jnp.take_along_axis(
        jnp.cumsum(one_hot, axis=0) - one_hot, species[:, None].astype(jnp.int32), axis=1
    )[:, 0]
    nblk = (counts + BM - 1) // BM
    blk_end = jnp.cumsum(nblk)
    padded_start = (blk_end - nblk) * BM
    pos = (padded_start[species] + within).astype(jnp.int32)
    src = jnp.zeros((R,), jnp.int32).at[pos].set(jnp.arange(N, dtype=jnp.int32))
    g = jnp.arange(G, dtype=jnp.int32)
    e_of_g = jnp.minimum(
        jnp.searchsorted(blk_end, g, side="right"), E - 1
    ).astype(jnp.int32)
    return e_of_g, src, pos


def _chunk_rows(b_per_w, row_bytes, budget=230 * 1024):
    cmax = max(8, min(b_per_w, budget // row_bytes))
    for c in range(cmax - cmax % 8, 0, -8):
        if b_per_w % c == 0:
            return c
    return 8


def _sc_gather(table, idx, R):
    _, D = table.shape
    b_per_w = R // _NW
    C = _chunk_rows(b_per_w, D * 4)
    NCH = b_per_w // C
    mesh = plsc.VectorSubcoreMesh(core_axis_name="c", subcore_axis_name="s")

    @functools.partial(
        pl.kernel,
        mesh=mesh,
        out_type=jax.ShapeDtypeStruct((R, D), table.dtype),
        scratch_types=[
            pltpu.VMEM((C,), jnp.int32),
            pltpu.VMEM((C,), jnp.int32),
            pltpu.VMEM((C, D), table.dtype),
            pltpu.VMEM((C, D), table.dtype),
            pltpu.SemaphoreType.DMA,
            pltpu.SemaphoreType.DMA,
        ],
    )
    def gather_k(table_hbm, idx_hbm, out_hbm, i0, i1, r0, r1, gsem, wsem):
        wid = lax.axis_index("s") * _SC_CORES + lax.axis_index("c")
        base = wid * b_per_w
        idx_v = (i0, i1)
        rows_v = (r0, r1)
        gathers = [None] * NCH
        writes = [None] * NCH
        pltpu.sync_copy(idx_hbm.at[pl.ds(base, C)], idx_v[0])
        gathers[0] = pltpu.async_copy(table_hbm.at[idx_v[0]], rows_v[0], gsem)
        for j in range(NCH):
            if j + 1 < NCH:
                if j >= 1:
                    writes[j - 1].wait()
                pltpu.sync_copy(
                    idx_hbm.at[pl.ds(base + (j + 1) * C, C)], idx_v[(j + 1) % 2]
                )
                gathers[j + 1] = pltpu.async_copy(
                    table_hbm.at[idx_v[(j + 1) % 2]], rows_v[(j + 1) % 2], gsem
                )
            gathers[j].wait()
            writes[j] = pltpu.async_copy(
                rows_v[j % 2], out_hbm.at[pl.ds(base + j * C, C)], wsem
            )
        for j in range(max(0, NCH - 2), NCH):
            writes[j].wait()

    return gather_k(table, idx)


def _mlp_body(e_ref, x_ref, w1_ref, b1_ref, w2_ref, b2_ref, y_ref):
    xb = x_ref[...].astype(jnp.bfloat16)
    h = jnp.dot(xb, w1_ref[0], preferred_element_type=jnp.float32)
    h = jax.nn.silu(h + b1_ref[0, 0])
    y = jnp.dot(h.astype(jnp.bfloat16), w2_ref[0], preferred_element_type=jnp.float32)
    y_ref[...] = y + b2_ref[0, 0]


def _grouped_mlp(e_of_g, x, W1, b1, W2, b2, BM, G):
    R = x.shape[0]
    D_IN, D_H = W1.shape[1], W1.shape[2]
    D_OUT = W2.shape[2]
    grid_spec = pltpu.PrefetchScalarGridSpec(
        num_scalar_prefetch=1,
        grid=(G,),
        in_specs=[
            pl.BlockSpec((BM, D_IN), lambda g, e: (g, 0)),
            pl.BlockSpec((1, D_IN, D_H), lambda g, e: (e[g], 0, 0)),
            pl.BlockSpec((1, 1, D_H), lambda g, e: (e[g], 0, 0)),
            pl.BlockSpec((1, D_H, D_OUT), lambda g, e: (e[g], 0, 0)),
            pl.BlockSpec((1, 1, D_OUT), lambda g, e: (e[g], 0, 0)),
        ],
        out_specs=pl.BlockSpec((BM, D_OUT), lambda g, e: (g, 0)),
    )
    return pl.pallas_call(
        _mlp_body,
        grid_spec=grid_spec,
        out_shape=jax.ShapeDtypeStruct((R, D_OUT), jnp.float32),
    )(e_of_g, x, W1, b1[:, None, :], W2, b2[:, None, :])


def kernel(density, species, W1, b1, W2, b2):
    N, D_IN = density.shape
    E = W1.shape[0]
    BM = 128
    G = N // BM + E
    R = G * BM

    e_of_g, src, pos = _routing(species, N, E, BM, G)
    gathered = _sc_gather(density, src, R)
    y = _grouped_mlp(
        e_of_g,
        gathered,
        W1.astype(jnp.bfloat16),
        b1,
        W2.astype(jnp.bfloat16),
        b2,
        BM,
        G,
    )
    return _sc_gather(y, pos, N)
